# baseline (speedup 1.0000x reference)
import jax
import jax.numpy as jnp
from jax import lax
from jax.experimental import pallas as pl
from jax.experimental.pallas import tpu as pltpu
from jax.experimental.pallas import tpu_sc as plsc

N = 10000
NP = 10240
F = 128
HID = 32
P = 12

NCHUNK = 12
CW = 64
NSC = 2
NSUB = 16
LANES = 16

EB = 96
NB = 216
EPT = EB * NB
ET = EPT * NSUB
RING = 2

ROWS_PER_SUB = NP // NSUB
CHUNKS_PER_CORE = NCHUNK // NSC


BN1 = 2000


def _proj_body(x_ref, wz_ref, wh_ref, wlz_ref, wlh_ref, u_ref):
    wz2 = jnp.dot(wz_ref[...], wlz_ref[:HID, :],
                  preferred_element_type=jnp.float32)
    wh2 = jnp.dot(wh_ref[...], wlh_ref[:HID, :],
                  preferred_element_type=jnp.float32)
    wcat = jnp.concatenate([wz2, wh2], axis=1)
    xb = x_ref[...]
    for t in range(P):
        wbig = jnp.pad(wcat[:, None, :], ((0, 0), (t, P - 1 - t), (0, 0)))
        wbig = wbig.reshape(F * P, CW)
        u_ref[t] = jnp.dot(xb, wbig,
                           preferred_element_type=jnp.float32
                           ).astype(jnp.bfloat16)


def _project(x2, Wz, Wh, Wlz, Wlh):
    return pl.pallas_call(
        _proj_body,
        grid=(N // BN1,),
        in_specs=[
            pl.BlockSpec((BN1, F * P), lambda nb: (nb, 0)),
            pl.BlockSpec((F, HID), lambda nb: (0, 0)),
            pl.BlockSpec((F, HID), lambda nb: (0, 0)),
            pl.BlockSpec((2 * HID, HID), lambda nb: (0, 0)),
            pl.BlockSpec((2 * HID, HID), lambda nb: (0, 0)),
        ],
        out_specs=pl.BlockSpec((NCHUNK, BN1, CW), lambda nb: (0, nb, 0)),
        out_shape=jax.ShapeDtypeStruct((NCHUNK, N, CW), jnp.bfloat16),
    )(x2, Wz, Wh, Wlz, Wlh)


def _agg_body(u_hbm, src_hbm, dst_hbm, w_hbm, out_hbm,
              src_v, dst_v, w_v,
              stg0, stg1, obuf,
              dinv_v, zdbuf,
              deg_sp, agg_sp,
              gs0, gs1, ss0):
    c = lax.axis_index("c")
    s = lax.axis_index("s")
    rbase = s * ROWS_PER_SUB

    stg = [stg0, stg1]
    gsems = [gs0, gs1]

    zeros16 = jnp.zeros((LANES,), jnp.float32)

    pltpu.sync_copy(src_hbm.at[s], src_v)
    pltpu.sync_copy(dst_hbm.at[s], dst_v)
    pltpu.sync_copy(w_hbm.at[s], w_v)

    def _zero_zd(i, carry):
        zdbuf[pl.ds(i * LANES, LANES)] = zeros16
        return carry
    lax.fori_loop(0, ROWS_PER_SUB // LANES, _zero_zd, 0)
    pltpu.sync_copy(zdbuf, deg_sp.at[pl.ds(rbase, ROWS_PER_SUB)])
    plsc.subcore_barrier()

    def _deg_acc(g, carry):
        for q in range(8):
            pltpu.async_copy(w_v.at[g * 8 + q],
                             deg_sp.at[dst_v.at[g * 8 + q]], ss0, add=True)
        for q in range(8):
            pltpu.make_async_copy(w_v.at[0], deg_sp.at[dst_v.at[0]],
                                  ss0).wait()
        return carry
    lax.fori_loop(0, NB // 8, _deg_acc, 0)
    plsc.subcore_barrier()

    pltpu.sync_copy(deg_sp.at[pl.ds(rbase, ROWS_PER_SUB)],
                    dinv_v.at[pl.ds(0, ROWS_PER_SUB)])
    magic = jnp.full((LANES,), 0x5F3759DF, jnp.int32)

    def _newton(i, carry):
        d = dinv_v[pl.ds(i * LANES, LANES)]
        iy = magic - lax.shift_right_logical(plsc.bitcast(d, jnp.int32), 1)
        y = plsc.bitcast(iy, jnp.float32)
        hd = d * 0.5
        for _ in range(3):
            y = y * (1.5 - hd * y * y)
        zdbuf[pl.ds(i * LANES, LANES)] = y
        return carry
    lax.fori_loop(0, ROWS_PER_SUB // LANES, _newton, 0)
    pltpu.sync_copy(zdbuf, deg_sp.at[pl.ds(rbase, ROWS_PER_SUB)])
    plsc.subcore_barrier()

    pltpu.sync_copy(deg_sp, dinv_v)

    def _norm(j, carry):
        for k in range(EB // LANES):
            sl = pl.ds(k * LANES, LANES)
            sv = src_v[j, sl]
            dv = dst_v[j, sl]
            wv = w_v[j, sl]
            a = plsc.load_gather(dinv_v, [sv])
            b = plsc.load_gather(dinv_v, [dv])
            w_v[j, sl] = a * wv * b
        return carry
    lax.fori_loop(0, NB, _norm, 0)

    def _fire_gather(j, r, cc):
        pltpu.async_copy(u_hbm.at[cc].at[src_v.at[j]], stg[r], gsems[r])

    def _wait_gather(r, cc):
        pltpu.make_async_copy(u_hbm.at[cc].at[src_v.at[0]], stg[r],
                              gsems[r]).wait()

    def _do_scatter(j):
        pltpu.async_copy(obuf, agg_sp.at[dst_v.at[j]], ss0, add=True)

    def _wait_scatter():
        pltpu.make_async_copy(obuf, agg_sp.at[dst_v.at[0]], ss0).wait()

    def _process(j, r):
        @plsc.parallel_loop(0, EB, 1, unroll=4)
        def _edge(i):
            nsp = plsc.load_gather(
                w_v, [jnp.broadcast_to(j, (LANES,)).astype(jnp.int32),
                      jnp.broadcast_to(i, (LANES,)).astype(jnp.int32)])
            for m in range(CW // (2 * LANES)):
                x = stg[r][i, pl.ds(m * 2 * LANES, 2 * LANES)]
                a, b = plsc.unpack(x, format=plsc.PackFormat.INTERLEAVED)
                obuf[i, pl.ds(m * 2 * LANES, LANES)] = a * nsp
                obuf[i, pl.ds(m * 2 * LANES + LANES, LANES)] = b * nsp

    def _turn(j, r, cc, do_fire=True):
        _wait_gather(r, cc)
        _process(j, r)
        if do_fire:
            _fire_gather(j + 2, r, cc)
        _do_scatter(j)
        _wait_scatter()

    for cci in range(CHUNKS_PER_CORE):
        cc = (c * CHUNKS_PER_CORE + cci).astype(jnp.int32)

        def _zero_stg(i, carry):
            for m in range(CW // LANES):
                obuf[i, pl.ds(m * LANES, LANES)] = zeros16
            return carry
        lax.fori_loop(0, EB, _zero_stg, 0)
        tail = ROWS_PER_SUB - (ROWS_PER_SUB // EB) * EB
        for m in range(ROWS_PER_SUB // EB):
            pltpu.sync_copy(obuf, agg_sp.at[pl.ds(rbase + m * EB, EB)])
        if tail:
            pltpu.sync_copy(
                obuf.at[pl.ds(0, tail)],
                agg_sp.at[pl.ds(rbase + (ROWS_PER_SUB // EB) * EB, tail)])
        plsc.subcore_barrier()

        _fire_gather(0, 0, cc)
        _fire_gather(1, 1, cc)

        def _group(g, carry):
            j0 = g * RING
            for rr in range(RING):
                _turn(j0 + rr, rr, cc)
            return carry
        lax.fori_loop(0, (NB - 2) // RING, _group, 0)

        for j in (NB - 2, NB - 1):
            _turn(j, j % RING, cc, do_fire=False)
        plsc.subcore_barrier()

        pltpu.sync_copy(agg_sp.at[pl.ds(rbase, ROWS_PER_SUB)],
                        out_hbm.at[cc, pl.ds(rbase, ROWS_PER_SUB)])
        plsc.subcore_barrier()


_AGG_CACHE = []


def _aggregate(U2, src_f, dst_f, w_f):
    if not _AGG_CACHE:
        _AGG_CACHE.append(pl.kernel(
            _agg_body,
            out_type=jax.ShapeDtypeStruct((NCHUNK, NP, CW), jnp.float32),
            mesh=plsc.VectorSubcoreMesh(
                core_axis_name="c", subcore_axis_name="s",
                num_cores=NSC, num_subcores=NSUB),
            scratch_types=_AGG_SCRATCH,
            compiler_params=pltpu.CompilerParams(
                needs_layout_passes=False, use_tc_tiling_on_sc=False)))
    return _AGG_CACHE[0](U2, src_f, dst_f, w_f)


_AGG_SCRATCH = [
        pltpu.VMEM((NB, EB), jnp.int32),
        pltpu.VMEM((NB, EB), jnp.int32),
        pltpu.VMEM((NB, EB), jnp.float32),
        pltpu.VMEM((EB, CW), jnp.bfloat16),
        pltpu.VMEM((EB, CW), jnp.bfloat16),
        pltpu.VMEM((EB, CW), jnp.float32),
        pltpu.VMEM((NP,), jnp.float32),
        pltpu.VMEM((ROWS_PER_SUB,), jnp.float32),
        pltpu.VMEM_SHARED((NP,), jnp.float32),
        pltpu.VMEM_SHARED((NP, CW), jnp.float32),
        pltpu.SemaphoreType.DMA,
        pltpu.SemaphoreType.DMA,
        pltpu.SemaphoreType.DMA,
]


BN4 = 2048


def _gate_body(agg_ref, bz_ref, bh_ref, blz_ref, blh_ref,
               wlz_ref, wlh_ref, att_ref, wl_ref, bl_ref, out_ref):
    bz2 = jnp.dot(bz_ref[...], wlz_ref[:HID, :],
                  preferred_element_type=jnp.float32) + blz_ref[...]
    bh2 = jnp.dot(bh_ref[...], wlh_ref[:HID, :],
                  preferred_element_type=jnp.float32) + blh_ref[...]
    att = att_ref[...]
    att = att - jnp.max(att, axis=1, keepdims=True)
    pexp = jnp.exp(att)
    probs = pexp / jnp.sum(pexp, axis=1, keepdims=True)
    acc = jnp.zeros((BN4, HID), jnp.float32)
    for t in range(P):
        chunk = agg_ref[t]
        prez = chunk[:, :HID] + bz2
        preh = chunk[:, HID:] + bh2
        zt = jax.nn.sigmoid(prez)
        ht = jnp.tanh(preh)
        acc = acc + probs[0, t] * ((1.0 - zt) * ht)
    h = jnp.maximum(acc, 0.0)
    out_ref[...] = jnp.dot(h, wl_ref[...],
                           preferred_element_type=jnp.float32) + bl_ref[...]


def _gates(agg, bz, bh, blz, blh, Wlz, Wlh, att2, Wl, bl2):
    return pl.pallas_call(
        _gate_body,
        grid=(NP // BN4,),
        in_specs=[
            pl.BlockSpec((NCHUNK, BN4, CW), lambda i: (0, i, 0)),
            pl.BlockSpec((1, HID), lambda i: (0, 0)),
            pl.BlockSpec((1, HID), lambda i: (0, 0)),
            pl.BlockSpec((1, HID), lambda i: (0, 0)),
            pl.BlockSpec((1, HID), lambda i: (0, 0)),
            pl.BlockSpec((2 * HID, HID), lambda i: (0, 0)),
            pl.BlockSpec((2 * HID, HID), lambda i: (0, 0)),
            pl.BlockSpec((1, P), lambda i: (0, 0)),
            pl.BlockSpec((HID, P), lambda i: (0, 0)),
            pl.BlockSpec((1, P), lambda i: (0, 0)),
        ],
        out_specs=pl.BlockSpec((BN4, P), lambda i: (i, 0)),
        out_shape=jax.ShapeDtypeStruct((NP, P), jnp.float32),
    )(agg, bz.reshape(1, HID), bh.reshape(1, HID), blz.reshape(1, HID),
      blh.reshape(1, HID), Wlz, Wlh, att2, Wl, bl2)


def kernel(x, edge_index, edge_attr, Wz, bz, Wr, br, Wh, bh,
           Wlz, blz, Wlr, blr, Wlh, blh, att, Wl, bl):
    f32 = jnp.float32
    e_in = edge_index.shape[1]

    x2 = x.reshape(N, F * P)

    src = edge_index[0]
    dst = edge_index[1]
    loop_idx = jnp.arange(N, dtype=src.dtype)
    padn = ET - (e_in + N)
    pad_ar = jnp.arange(padn, dtype=src.dtype)
    src_f = jnp.concatenate([src, loop_idx, pad_ar % N]).reshape(NSUB, NB, EB)
    dst_f = jnp.concatenate([dst, loop_idx,
                             N + pad_ar % (NP - N)]).reshape(NSUB, NB, EB)
    w_f = jnp.concatenate([edge_attr, jnp.ones((N,), f32),
                           jnp.zeros((padn,), f32)]).reshape(NSUB, NB, EB)

    perm = jnp.stack([jnp.arange(HID // 2), jnp.arange(HID // 2) + HID // 2],
                     axis=1).reshape(-1)
    U = _project(x2, Wz, Wh, Wlz[:, perm], Wlh[:, perm])
    agg = _aggregate(U, src_f, dst_f, w_f)
    out = _gates(agg, bz, bh, blz, blh, Wlz, Wlh,
                 att.reshape(1, P), Wl, bl.reshape(1, P))
    return out[:N]


# speedup vs baseline: 1.2773x; 1.2773x over previous
import jax
import jax.numpy as jnp
from jax import lax
from jax.experimental import pallas as pl
from jax.experimental.pallas import tpu as pltpu
from jax.experimental.pallas import tpu_sc as plsc

N = 10000
NP = 10240
F = 128
HID = 32
P = 12

NCHUNK = 12
CW = 64
NSC = 2
NSUB = 16
LANES = 16

EB = 128
NB = 162
EPT = EB * NB
ET = EPT * NSUB
RING = 2

ROWS_PER_SUB = NP // NSUB
CHUNKS_PER_CORE = NCHUNK // NSC


BN1 = 2000


def _proj_body(x_ref, wz_ref, wh_ref, wlz_ref, wlh_ref, u_ref):
    wz2 = jnp.dot(wz_ref[...], wlz_ref[:HID, :],
                  preferred_element_type=jnp.float32)
    wh2 = jnp.dot(wh_ref[...], wlh_ref[:HID, :],
                  preferred_element_type=jnp.float32)
    wcat = jnp.concatenate([wz2, wh2], axis=1)
    u_ref[0] = jnp.dot(x_ref[0], wcat,
                       preferred_element_type=jnp.float32).astype(jnp.bfloat16)


def _project(xT, Wz, Wh, Wlz, Wlh):
    return pl.pallas_call(
        _proj_body,
        grid=(P, N // BN1),
        in_specs=[
            pl.BlockSpec((1, BN1, F), lambda t, nb: (t, nb, 0)),
            pl.BlockSpec((F, HID), lambda t, nb: (0, 0)),
            pl.BlockSpec((F, HID), lambda t, nb: (0, 0)),
            pl.BlockSpec((2 * HID, HID), lambda t, nb: (0, 0)),
            pl.BlockSpec((2 * HID, HID), lambda t, nb: (0, 0)),
        ],
        out_specs=pl.BlockSpec((1, BN1, CW), lambda t, nb: (t, nb, 0)),
        out_shape=jax.ShapeDtypeStruct((NCHUNK, N, CW), jnp.bfloat16),
    )(xT, Wz, Wh, Wlz, Wlh)


def _agg_body(u_hbm, src_hbm, dst_hbm, w_hbm, out_hbm,
              src_v, dst_v, w_v,
              stg0, stg1, obuf,
              dinv_v, zdbuf,
              deg_sp, agg_sp,
              gs0, gs1, ss0):
    c = lax.axis_index("c")
    s = lax.axis_index("s")
    rbase = s * ROWS_PER_SUB

    stg = [stg0, stg1]
    gsems = [gs0, gs1]

    zeros16 = jnp.zeros((LANES,), jnp.float32)

    pltpu.sync_copy(src_hbm.at[s], src_v)
    pltpu.sync_copy(dst_hbm.at[s], dst_v)
    pltpu.sync_copy(w_hbm.at[s], w_v)

    def _zero_zd(i, carry):
        zdbuf[pl.ds(i * LANES, LANES)] = zeros16
        return carry
    lax.fori_loop(0, ROWS_PER_SUB // LANES, _zero_zd, 0)
    pltpu.sync_copy(zdbuf, deg_sp.at[pl.ds(rbase, ROWS_PER_SUB)])
    plsc.subcore_barrier()

    def _deg_acc(g, carry):
        for q in range(8):
            pltpu.async_copy(w_v.at[g * 8 + q],
                             deg_sp.at[dst_v.at[g * 8 + q]], ss0, add=True)
        for q in range(8):
            pltpu.make_async_copy(w_v.at[0], deg_sp.at[dst_v.at[0]],
                                  ss0).wait()
        return carry
    lax.fori_loop(0, NB // 8, _deg_acc, 0)
    plsc.subcore_barrier()

    pltpu.sync_copy(deg_sp.at[pl.ds(rbase, ROWS_PER_SUB)],
                    dinv_v.at[pl.ds(0, ROWS_PER_SUB)])
    magic = jnp.full((LANES,), 0x5F3759DF, jnp.int32)

    def _newton(i, carry):
        d = dinv_v[pl.ds(i * LANES, LANES)]
        iy = magic - lax.shift_right_logical(plsc.bitcast(d, jnp.int32), 1)
        y = plsc.bitcast(iy, jnp.float32)
        hd = d * 0.5
        for _ in range(3):
            y = y * (1.5 - hd * y * y)
        zdbuf[pl.ds(i * LANES, LANES)] = y
        return carry
    lax.fori_loop(0, ROWS_PER_SUB // LANES, _newton, 0)
    pltpu.sync_copy(zdbuf, deg_sp.at[pl.ds(rbase, ROWS_PER_SUB)])
    plsc.subcore_barrier()

    pltpu.sync_copy(deg_sp, dinv_v)

    def _norm(j, carry):
        for k in range(EB // LANES):
            sl = pl.ds(k * LANES, LANES)
            sv = src_v[j, sl]
            dv = dst_v[j, sl]
            wv = w_v[j, sl]
            a = plsc.load_gather(dinv_v, [sv])
            b = plsc.load_gather(dinv_v, [dv])
            w_v[j, sl] = a * wv * b
        return carry
    lax.fori_loop(0, NB, _norm, 0)

    def _fire_gather(j, r, cc):
        pltpu.async_copy(u_hbm.at[cc].at[src_v.at[j]], stg[r], gsems[r])

    def _wait_gather(r, cc):
        pltpu.make_async_copy(u_hbm.at[cc].at[src_v.at[0]], stg[r],
                              gsems[r]).wait()

    def _do_scatter(j):
        pltpu.async_copy(obuf, agg_sp.at[dst_v.at[j]], ss0, add=True)

    def _wait_scatter():
        pltpu.make_async_copy(obuf, agg_sp.at[dst_v.at[0]], ss0).wait()

    def _process(j, r):
        @plsc.parallel_loop(0, EB, 1, unroll=4)
        def _edge(i):
            nsp = plsc.load_gather(
                w_v, [jnp.broadcast_to(j, (LANES,)).astype(jnp.int32),
                      jnp.broadcast_to(i, (LANES,)).astype(jnp.int32)])
            for m in range(CW // (2 * LANES)):
                x = stg[r][i, pl.ds(m * 2 * LANES, 2 * LANES)]
                a, b = plsc.unpack(x, format=plsc.PackFormat.INTERLEAVED)
                obuf[i, pl.ds(m * 2 * LANES, LANES)] = a * nsp
                obuf[i, pl.ds(m * 2 * LANES + LANES, LANES)] = b * nsp

    def _turn(j, r, cc, do_fire=True):
        _wait_gather(r, cc)
        _process(j, r)
        if do_fire:
            _fire_gather(j + 2, r, cc)
        _do_scatter(j)
        _wait_scatter()

    for cci in range(CHUNKS_PER_CORE):
        cc = (c * CHUNKS_PER_CORE + cci).astype(jnp.int32)

        def _zero_stg(i, carry):
            for m in range(CW // LANES):
                obuf[i, pl.ds(m * LANES, LANES)] = zeros16
            return carry
        lax.fori_loop(0, EB, _zero_stg, 0)
        tail = ROWS_PER_SUB - (ROWS_PER_SUB // EB) * EB
        for m in range(ROWS_PER_SUB // EB):
            pltpu.sync_copy(obuf, agg_sp.at[pl.ds(rbase + m * EB, EB)])
        if tail:
            pltpu.sync_copy(
                obuf.at[pl.ds(0, tail)],
                agg_sp.at[pl.ds(rbase + (ROWS_PER_SUB // EB) * EB, tail)])
        plsc.subcore_barrier()

        _fire_gather(0, 0, cc)
        _fire_gather(1, 1, cc)

        def _group(g, carry):
            j0 = g * RING
            for rr in range(RING):
                _turn(j0 + rr, rr, cc)
            return carry
        lax.fori_loop(0, (NB - 2) // RING, _group, 0)

        for j in (NB - 2, NB - 1):
            _turn(j, j % RING, cc, do_fire=False)
        plsc.subcore_barrier()

        pltpu.sync_copy(agg_sp.at[pl.ds(rbase, ROWS_PER_SUB)],
                        out_hbm.at[cc, pl.ds(rbase, ROWS_PER_SUB)])
        plsc.subcore_barrier()


_AGG_CACHE = []


def _aggregate(U2, src_f, dst_f, w_f):
    if not _AGG_CACHE:
        _AGG_CACHE.append(pl.kernel(
            _agg_body,
            out_type=jax.ShapeDtypeStruct((NCHUNK, NP, CW), jnp.float32),
            mesh=plsc.VectorSubcoreMesh(
                core_axis_name="c", subcore_axis_name="s",
                num_cores=NSC, num_subcores=NSUB),
            scratch_types=_AGG_SCRATCH,
            compiler_params=pltpu.CompilerParams(
                needs_layout_passes=False, use_tc_tiling_on_sc=False)))
    return _AGG_CACHE[0](U2, src_f, dst_f, w_f)


_AGG_SCRATCH = [
        pltpu.VMEM((NB, EB), jnp.int32),
        pltpu.VMEM((NB, EB), jnp.int32),
        pltpu.VMEM((NB, EB), jnp.float32),
        pltpu.VMEM((EB, CW), jnp.bfloat16),
        pltpu.VMEM((EB, CW), jnp.bfloat16),
        pltpu.VMEM((EB, CW), jnp.float32),
        pltpu.VMEM((NP,), jnp.float32),
        pltpu.VMEM((ROWS_PER_SUB,), jnp.float32),
        pltpu.VMEM_SHARED((NP,), jnp.float32),
        pltpu.VMEM_SHARED((NP, CW), jnp.float32),
        pltpu.SemaphoreType.DMA,
        pltpu.SemaphoreType.DMA,
        pltpu.SemaphoreType.DMA,
]


BN4 = 2048


def _gate_body(agg_ref, bz_ref, bh_ref, blz_ref, blh_ref,
               wlz_ref, wlh_ref, att_ref, wl_ref, bl_ref, out_ref):
    bz2 = jnp.dot(bz_ref[...], wlz_ref[:HID, :],
                  preferred_element_type=jnp.float32) + blz_ref[...]
    bh2 = jnp.dot(bh_ref[...], wlh_ref[:HID, :],
                  preferred_element_type=jnp.float32) + blh_ref[...]
    att = att_ref[...]
    att = att - jnp.max(att, axis=1, keepdims=True)
    pexp = jnp.exp(att)
    probs = pexp / jnp.sum(pexp, axis=1, keepdims=True)
    acc = jnp.zeros((BN4, HID), jnp.float32)
    for t in range(P):
        chunk = agg_ref[t]
        prez = chunk[:, :HID] + bz2
        preh = chunk[:, HID:] + bh2
        zt = jax.nn.sigmoid(prez)
        ht = jnp.tanh(preh)
        acc = acc + probs[0, t] * ((1.0 - zt) * ht)
    h = jnp.maximum(acc, 0.0)
    out_ref[...] = jnp.dot(h, wl_ref[...],
                           preferred_element_type=jnp.float32) + bl_ref[...]


def _gates(agg, bz, bh, blz, blh, Wlz, Wlh, att2, Wl, bl2):
    return pl.pallas_call(
        _gate_body,
        grid=(NP // BN4,),
        in_specs=[
            pl.BlockSpec((NCHUNK, BN4, CW), lambda i: (0, i, 0)),
            pl.BlockSpec((1, HID), lambda i: (0, 0)),
            pl.BlockSpec((1, HID), lambda i: (0, 0)),
            pl.BlockSpec((1, HID), lambda i: (0, 0)),
            pl.BlockSpec((1, HID), lambda i: (0, 0)),
            pl.BlockSpec((2 * HID, HID), lambda i: (0, 0)),
            pl.BlockSpec((2 * HID, HID), lambda i: (0, 0)),
            pl.BlockSpec((1, P), lambda i: (0, 0)),
            pl.BlockSpec((HID, P), lambda i: (0, 0)),
            pl.BlockSpec((1, P), lambda i: (0, 0)),
        ],
        out_specs=pl.BlockSpec((BN4, P), lambda i: (i, 0)),
        out_shape=jax.ShapeDtypeStruct((NP, P), jnp.float32),
    )(agg, bz.reshape(1, HID), bh.reshape(1, HID), blz.reshape(1, HID),
      blh.reshape(1, HID), Wlz, Wlh, att2, Wl, bl2)


def kernel(x, edge_index, edge_attr, Wz, bz, Wr, br, Wh, bh,
           Wlz, blz, Wlr, blr, Wlh, blh, att, Wl, bl):
    f32 = jnp.float32
    e_in = edge_index.shape[1]

    xT = jnp.transpose(x, (2, 0, 1))

    src = edge_index[0]
    dst = edge_index[1]
    loop_idx = jnp.arange(N, dtype=src.dtype)
    padn = ET - (e_in + N)
    pad_ar = jnp.arange(padn, dtype=src.dtype)
    src_f = jnp.concatenate([src, loop_idx, pad_ar % N]).reshape(NSUB, NB, EB)
    dst_f = jnp.concatenate([dst, loop_idx,
                             N + pad_ar % (NP - N)]).reshape(NSUB, NB, EB)
    w_f = jnp.concatenate([edge_attr, jnp.ones((N,), f32),
                           jnp.zeros((padn,), f32)]).reshape(NSUB, NB, EB)

    perm = jnp.stack([jnp.arange(HID // 2), jnp.arange(HID // 2) + HID // 2],
                     axis=1).reshape(-1)
    U = _project(xT, Wz, Wh, Wlz[:, perm], Wlh[:, perm])
    agg = _aggregate(U, src_f, dst_f, w_f)
    out = _gates(agg, bz, bh, blz, blh, Wlz, Wlh,
                 att.reshape(1, P), Wl, bl.reshape(1, P))
    return out[:N]
